# Initial kernel scaffold; baseline (speedup 1.0000x reference)
import jax
import jax.numpy as jnp
from jax.experimental import pallas as pl


def kernel(x, edge_index, W1q, b1q, W1k, b1k, W1v, b1v, W1s, b1s, W2q, b2q, W2k, b2k, W2v, b2v, W2s, b2s):
    raise NotImplementedError("write your pallas kernel here")


# speedup vs baseline: 5.8243x; 5.8243x over previous
import functools

import jax
import jax.numpy as jnp
from jax import lax
from jax.experimental import pallas as pl
from jax.experimental.pallas import tpu as pltpu
from jax.experimental.pallas import tpu_sc as plsc

_N = 10000
_E = 320000
_D = 128
_NC = 2
_NS = 16
_NW = _NC * _NS
_EPW = _E // _NW
_C = 80
_NG = _C // 16
_NCHUNK = _EPW // _C
_NP = 10240
_RPT = _NP // _NS
_INV_SQRT_D = 1.0 / (_D ** 0.5)

_BN = 640
_GRID = _NP // _BN


def _proj_body(x_ref, w_ref, b_ref, q_ref, k_ref, v_ref, s_ref):
    acc = jnp.dot(x_ref[...], w_ref[...],
                  preferred_element_type=jnp.float32) + b_ref[...]
    q_ref[...] = acc[:, :_D]
    k_ref[...] = acc[:, _D:2 * _D]
    v_ref[...] = acc[:, 2 * _D:3 * _D]
    s_ref[...] = acc[:, 3 * _D:]


def _qkvs_specs():
    return dict(
        out_specs=[
            pl.BlockSpec((_BN, _D), lambda i: (i, 0)),
            pl.BlockSpec((_BN, _D), lambda i: (i, 0)),
            pl.BlockSpec((_BN, _D), lambda i: (i, 0)),
            pl.BlockSpec((_BN, _D), lambda i: (i, 0)),
        ],
        out_shape=[
            jax.ShapeDtypeStruct((_NP, _D), jnp.float32),
            jax.ShapeDtypeStruct((_NP, _D), jnp.float32),
            jax.ShapeDtypeStruct((_NP, _D), jnp.float32),
            jax.ShapeDtypeStruct((_NP, _D), jnp.float32),
        ],
    )


def _proj(x, w, b):
    return pl.pallas_call(
        _proj_body,
        grid=(_GRID,),
        in_specs=[
            pl.BlockSpec((_BN, _D), lambda i: (i, 0)),
            pl.BlockSpec((_D, 4 * _D), lambda i: (0, 0)),
            pl.BlockSpec((1, 4 * _D), lambda i: (0, 0)),
        ],
        **_qkvs_specs(),
    )(x, w, b)


def _combine_h(p_ref, dp_ref, s_ref, ones_ref):
    num = p_ref[0] + p_ref[1]
    dsum = dp_ref[0] + dp_ref[1]
    den = lax.dot_general(dsum, ones_ref[...],
                          (((0,), (0,)), ((), ())),
                          preferred_element_type=jnp.float32)
    return num / (den + 1e-16) + s_ref[...]


def _comb_proj_body(p_ref, dp_ref, s1_ref, ones_ref, w_ref, b_ref,
                    q_ref, k_ref, v_ref, s_ref):
    h = jnp.maximum(_combine_h(p_ref, dp_ref, s1_ref, ones_ref), 0.0)
    acc = jnp.dot(h, w_ref[...],
                  preferred_element_type=jnp.float32) + b_ref[...]
    q_ref[...] = acc[:, :_D]
    k_ref[...] = acc[:, _D:2 * _D]
    v_ref[...] = acc[:, 2 * _D:3 * _D]
    s_ref[...] = acc[:, 3 * _D:]


def _comb_proj(p, dp, s1, ones, w, b):
    return pl.pallas_call(
        _comb_proj_body,
        grid=(_GRID,),
        in_specs=[
            pl.BlockSpec((2, _BN, _D), lambda i: (0, i, 0)),
            pl.BlockSpec((2, _NS, _BN), lambda i: (0, 0, i)),
            pl.BlockSpec((_BN, _D), lambda i: (i, 0)),
            pl.BlockSpec((_NS, 1), lambda i: (0, 0)),
            pl.BlockSpec((_D, 4 * _D), lambda i: (0, 0)),
            pl.BlockSpec((1, 4 * _D), lambda i: (0, 0)),
        ],
        **_qkvs_specs(),
    )(p, dp, s1, ones, w, b)


def _comb_body(p_ref, dp_ref, s2_ref, ones_ref, o_ref):
    o_ref[...] = _combine_h(p_ref, dp_ref, s2_ref, ones_ref)


def _comb(p, dp, s2, ones):
    return pl.pallas_call(
        _comb_body,
        grid=(_GRID,),
        in_specs=[
            pl.BlockSpec((2, _BN, _D), lambda i: (0, i, 0)),
            pl.BlockSpec((2, _NS, _BN), lambda i: (0, 0, i)),
            pl.BlockSpec((_BN, _D), lambda i: (i, 0)),
            pl.BlockSpec((_NS, 1), lambda i: (0, 0)),
        ],
        out_specs=pl.BlockSpec((_BN, _D), lambda i: (i, 0)),
        out_shape=jax.ShapeDtypeStruct((_NP, _D), jnp.float32),
    )(p, dp, s2, ones)


def _edge_body(q_hbm, k_hbm, v_hbm, src_hbm, dst_hbm, zn_hbm, zd_hbm,
               part_hbm, dpart_hbm,
               srcv, dstv, qbuf, kbuf, vbuf, ebuf, dtile,
               acc, sem1, sem2, sem3):
    cid = lax.axis_index("c")
    sid = lax.axis_index("s")
    wid = sid * _NC + cid

    pltpu.sync_copy(zn_hbm, acc.at[pl.ds(sid * _RPT, _RPT)])
    pltpu.sync_copy(zd_hbm, dtile)
    plsc.subcore_barrier()

    lanes = lax.iota(jnp.int32, 16)
    base_w = wid * _EPW

    def chunk(ci, carry):
        base = base_w + ci * _C
        pltpu.sync_copy(src_hbm.at[pl.ds(base, _C)], srcv)
        pltpu.sync_copy(dst_hbm.at[pl.ds(base, _C)], dstv)
        cp1 = pltpu.async_copy(k_hbm.at[srcv], kbuf, sem1)
        cp2 = pltpu.async_copy(v_hbm.at[srcv], vbuf, sem2)
        cp3 = pltpu.async_copy(q_hbm.at[dstv], qbuf, sem3)
        cp1.wait()
        cp2.wait()
        cp3.wait()

        def group(g, carry2):
            def edge(ei, carry3):
                e = g * 16 + ei
                a16 = qbuf[e, pl.ds(0, 16)] * kbuf[e, pl.ds(0, 16)]
                for j in range(1, 8):
                    a16 = a16 + (qbuf[e, pl.ds(16 * j, 16)]
                                 * kbuf[e, pl.ds(16 * j, 16)])
                s = jnp.sum(a16) * _INV_SQRT_D
                ex16 = jnp.exp(jnp.full((16,), s, jnp.float32))
                for j in range(8):
                    vbuf[e, pl.ds(16 * j, 16)] = (
                        vbuf[e, pl.ds(16 * j, 16)] * ex16)
                ebuf[ei, pl.ds(0, 16)] = ex16
                return carry3

            lax.fori_loop(0, 16, edge, 0)
            exd = plsc.load_gather(ebuf, [lanes, lanes])
            dst16 = dstv[pl.ds(g * 16, 16)]
            plsc.addupdate_scatter(dtile, [dst16], exd)
            return carry2

        lax.fori_loop(0, _NG, group, 0)
        pltpu.sync_copy(vbuf, acc.at[dstv], add=True)
        return carry

    lax.fori_loop(0, _NCHUNK, chunk, 0)
    plsc.subcore_barrier()
    pltpu.sync_copy(acc.at[pl.ds(sid * _RPT, _RPT)],
                    part_hbm.at[cid, pl.ds(sid * _RPT, _RPT)])
    pltpu.sync_copy(dtile, dpart_hbm.at[cid, sid])


@functools.cache
def _make_edge():
    mesh = plsc.VectorSubcoreMesh(
        core_axis_name="c", subcore_axis_name="s",
        num_cores=_NC, num_subcores=_NS)
    return pl.kernel(
        _edge_body,
        out_type=(jax.ShapeDtypeStruct((_NC, _NP, _D), jnp.float32),
                  jax.ShapeDtypeStruct((_NC, _NS, _NP), jnp.float32)),
        mesh=mesh,
        compiler_params=pltpu.CompilerParams(needs_layout_passes=False),
        scratch_types=[
            pltpu.VMEM((_C,), jnp.int32),
            pltpu.VMEM((_C,), jnp.int32),
            pltpu.VMEM((_C, _D), jnp.float32),
            pltpu.VMEM((_C, _D), jnp.float32),
            pltpu.VMEM((_C, _D), jnp.float32),
            pltpu.VMEM((16, 16), jnp.float32),
            pltpu.VMEM((_NP,), jnp.float32),
            pltpu.VMEM_SHARED((_NP, _D), jnp.float32),
            pltpu.SemaphoreType.DMA,
            pltpu.SemaphoreType.DMA,
            pltpu.SemaphoreType.DMA,
        ],
    )


def _edge(q, k, v, src, dst, zn, zd):
    return _make_edge()(q, k, v, src, dst, zn, zd)


def kernel(x, edge_index, W1q, b1q, W1k, b1k, W1v, b1v, W1s, b1s,
           W2q, b2q, W2k, b2k, W2v, b2v, W2s, b2s):
    src = edge_index[0]
    dst = edge_index[1]
    w1 = jnp.concatenate([W1q.T, W1k.T, W1v.T, W1s.T], axis=1)
    b1 = jnp.concatenate([b1q, b1k, b1v, b1s])[None, :]
    w2 = jnp.concatenate([W2q.T, W2k.T, W2v.T, W2s.T], axis=1)
    b2 = jnp.concatenate([b2q, b2k, b2v, b2s])[None, :]
    xp = jnp.zeros((_NP, _D), jnp.float32).at[:_N].set(x)
    zn = jnp.zeros((_RPT, _D), jnp.float32)
    zd = jnp.zeros((_NP,), jnp.float32)
    ones = jnp.ones((_NS, 1), jnp.float32)

    q1, k1, v1, s1 = _proj(xp, w1, b1)
    p1, dp1 = _edge(q1, k1, v1, src, dst, zn, zd)
    q2, k2, v2, s2 = _comb_proj(p1, dp1, s1, ones, w2, b2)
    p2, dp2 = _edge(q2, k2, v2, src, dst, zn, zd)
    return _comb(p2, dp2, s2, ones)[:_N]


# speedup vs baseline: 8.5037x; 1.4600x over previous
import functools

import jax
import jax.numpy as jnp
from jax import lax
from jax.experimental import pallas as pl
from jax.experimental.pallas import tpu as pltpu
from jax.experimental.pallas import tpu_sc as plsc

_N = 10000
_E = 320000
_D = 128
_NC = 2
_NS = 16
_NW = _NC * _NS
_EPW = _E // _NW
_C = 80
_NG = _C // 16
_NCHUNK = _EPW // _C
_NP = 10240
_RPT = _NP // _NS
_INV_SQRT_D = 1.0 / (_D ** 0.5)

_BN = 640
_GRID = _NP // _BN


def _proj_body(x_ref, w_ref, b_ref, q_ref, k_ref, v_ref, s_ref):
    acc = jnp.dot(x_ref[...], w_ref[...],
                  preferred_element_type=jnp.float32) + b_ref[...]
    q_ref[...] = acc[:, :_D]
    k_ref[...] = acc[:, _D:2 * _D]
    v_ref[...] = acc[:, 2 * _D:3 * _D]
    s_ref[...] = acc[:, 3 * _D:]


def _qkvs_specs():
    return dict(
        out_specs=[
            pl.BlockSpec((_BN, _D), lambda i: (i, 0)),
            pl.BlockSpec((_BN, _D), lambda i: (i, 0)),
            pl.BlockSpec((_BN, _D), lambda i: (i, 0)),
            pl.BlockSpec((_BN, _D), lambda i: (i, 0)),
        ],
        out_shape=[
            jax.ShapeDtypeStruct((_NP, _D), jnp.float32),
            jax.ShapeDtypeStruct((_NP, _D), jnp.float32),
            jax.ShapeDtypeStruct((_NP, _D), jnp.float32),
            jax.ShapeDtypeStruct((_NP, _D), jnp.float32),
        ],
    )


def _proj(x, w, b):
    return pl.pallas_call(
        _proj_body,
        grid=(_GRID,),
        in_specs=[
            pl.BlockSpec((_BN, _D), lambda i: (i, 0)),
            pl.BlockSpec((_D, 4 * _D), lambda i: (0, 0)),
            pl.BlockSpec((1, 4 * _D), lambda i: (0, 0)),
        ],
        **_qkvs_specs(),
    )(x, w, b)


def _combine_h(p_ref, dp_ref, s_ref, ones_ref):
    num = p_ref[0] + p_ref[1]
    dsum = dp_ref[0] + dp_ref[1]
    den = lax.dot_general(dsum, ones_ref[...],
                          (((0,), (0,)), ((), ())),
                          preferred_element_type=jnp.float32)
    return num / (den + 1e-16) + s_ref[...]


def _comb_proj_body(p_ref, dp_ref, s1_ref, ones_ref, w_ref, b_ref,
                    q_ref, k_ref, v_ref, s_ref):
    h = jnp.maximum(_combine_h(p_ref, dp_ref, s1_ref, ones_ref), 0.0)
    acc = jnp.dot(h, w_ref[...],
                  preferred_element_type=jnp.float32) + b_ref[...]
    q_ref[...] = acc[:, :_D]
    k_ref[...] = acc[:, _D:2 * _D]
    v_ref[...] = acc[:, 2 * _D:3 * _D]
    s_ref[...] = acc[:, 3 * _D:]


def _comb_proj(p, dp, s1, ones, w, b):
    return pl.pallas_call(
        _comb_proj_body,
        grid=(_GRID,),
        in_specs=[
            pl.BlockSpec((2, _BN, _D), lambda i: (0, i, 0)),
            pl.BlockSpec((2, _NS, _BN), lambda i: (0, 0, i)),
            pl.BlockSpec((_BN, _D), lambda i: (i, 0)),
            pl.BlockSpec((_NS, 1), lambda i: (0, 0)),
            pl.BlockSpec((_D, 4 * _D), lambda i: (0, 0)),
            pl.BlockSpec((1, 4 * _D), lambda i: (0, 0)),
        ],
        **_qkvs_specs(),
    )(p, dp, s1, ones, w, b)


def _comb_body(p_ref, dp_ref, s2_ref, ones_ref, o_ref):
    o_ref[...] = _combine_h(p_ref, dp_ref, s2_ref, ones_ref)


def _comb(p, dp, s2, ones):
    return pl.pallas_call(
        _comb_body,
        grid=(_GRID,),
        in_specs=[
            pl.BlockSpec((2, _BN, _D), lambda i: (0, i, 0)),
            pl.BlockSpec((2, _NS, _BN), lambda i: (0, 0, i)),
            pl.BlockSpec((_BN, _D), lambda i: (i, 0)),
            pl.BlockSpec((_NS, 1), lambda i: (0, 0)),
        ],
        out_specs=pl.BlockSpec((_BN, _D), lambda i: (i, 0)),
        out_shape=jax.ShapeDtypeStruct((_NP, _D), jnp.float32),
    )(p, dp, s2, ones)


def _edge_body(q_hbm, k_hbm, v_hbm, src_hbm, dst_hbm, zn_hbm, zd_hbm,
               part_hbm, dpart_hbm,
               srcv, dstv, qbuf, kbuf, vbuf, ebuf, evec, dtile,
               acc, sem1, sem2, sem3):
    cid = lax.axis_index("c")
    sid = lax.axis_index("s")
    wid = sid * _NC + cid

    pltpu.sync_copy(zn_hbm, acc.at[pl.ds(sid * _RPT, _RPT)])
    pltpu.sync_copy(zd_hbm, dtile)
    plsc.subcore_barrier()

    lanes = lax.iota(jnp.int32, 16)
    base_w = wid * _EPW

    def chunk(ci, carry):
        base = base_w + ci * _C
        pltpu.sync_copy(src_hbm.at[pl.ds(base, _C)], srcv)
        pltpu.sync_copy(dst_hbm.at[pl.ds(base, _C)], dstv)
        cp1 = pltpu.async_copy(k_hbm.at[srcv], kbuf, sem1)
        cp2 = pltpu.async_copy(v_hbm.at[srcv], vbuf, sem2)
        cp3 = pltpu.async_copy(q_hbm.at[dstv], qbuf, sem3)
        cp1.wait()
        cp2.wait()
        cp3.wait()

        for g in range(_NG):
            off = g * 16

            def p1(ei, carry3):
                e = off + ei
                p0 = qbuf[e, pl.ds(0, 16)] * kbuf[e, pl.ds(0, 16)]
                p1_ = qbuf[e, pl.ds(16, 16)] * kbuf[e, pl.ds(16, 16)]
                p2 = qbuf[e, pl.ds(32, 16)] * kbuf[e, pl.ds(32, 16)]
                p3_ = qbuf[e, pl.ds(48, 16)] * kbuf[e, pl.ds(48, 16)]
                p4 = qbuf[e, pl.ds(64, 16)] * kbuf[e, pl.ds(64, 16)]
                p5 = qbuf[e, pl.ds(80, 16)] * kbuf[e, pl.ds(80, 16)]
                p6 = qbuf[e, pl.ds(96, 16)] * kbuf[e, pl.ds(96, 16)]
                p7 = qbuf[e, pl.ds(112, 16)] * kbuf[e, pl.ds(112, 16)]
                ebuf[ei, pl.ds(0, 16)] = (
                    ((p0 + p1_) + (p2 + p3_)) + ((p4 + p5) + (p6 + p7)))
                return carry3

            lax.fori_loop(0, 16, p1, 0, unroll=4)

            cols = [plsc.load_gather(
                ebuf, [lanes, jnp.full((16,), l, jnp.int32)])
                for l in range(16)]
            t0 = ((cols[0] + cols[1]) + (cols[2] + cols[3]))
            t1 = ((cols[4] + cols[5]) + (cols[6] + cols[7]))
            t2 = ((cols[8] + cols[9]) + (cols[10] + cols[11]))
            t3 = ((cols[12] + cols[13]) + (cols[14] + cols[15]))
            exv = jnp.exp(((t0 + t1) + (t2 + t3)) * _INV_SQRT_D)
            evec[pl.ds(0, 16)] = exv
            plsc.addupdate_scatter(dtile, [dstv[pl.ds(off, 16)]], exv)

            def p3(ei, carry3):
                e = off + ei
                exb = plsc.load_gather(
                    evec, [jnp.full((16,), ei, jnp.int32)])
                for j in range(8):
                    vbuf[e, pl.ds(16 * j, 16)] = (
                        vbuf[e, pl.ds(16 * j, 16)] * exb)
                return carry3

            lax.fori_loop(0, 16, p3, 0, unroll=4)

        pltpu.sync_copy(vbuf, acc.at[dstv], add=True)
        return carry

    lax.fori_loop(0, _NCHUNK, chunk, 0)
    plsc.subcore_barrier()
    pltpu.sync_copy(acc.at[pl.ds(sid * _RPT, _RPT)],
                    part_hbm.at[cid, pl.ds(sid * _RPT, _RPT)])
    pltpu.sync_copy(dtile, dpart_hbm.at[cid, sid])


@functools.cache
def _make_edge():
    mesh = plsc.VectorSubcoreMesh(
        core_axis_name="c", subcore_axis_name="s",
        num_cores=_NC, num_subcores=_NS)
    return pl.kernel(
        _edge_body,
        out_type=(jax.ShapeDtypeStruct((_NC, _NP, _D), jnp.float32),
                  jax.ShapeDtypeStruct((_NC, _NS, _NP), jnp.float32)),
        mesh=mesh,
        compiler_params=pltpu.CompilerParams(needs_layout_passes=False),
        scratch_types=[
            pltpu.VMEM((_C,), jnp.int32),
            pltpu.VMEM((_C,), jnp.int32),
            pltpu.VMEM((_C, _D), jnp.float32),
            pltpu.VMEM((_C, _D), jnp.float32),
            pltpu.VMEM((_C, _D), jnp.float32),
            pltpu.VMEM((16, 16), jnp.float32),
            pltpu.VMEM((16,), jnp.float32),
            pltpu.VMEM((_NP,), jnp.float32),
            pltpu.VMEM_SHARED((_NP, _D), jnp.float32),
            pltpu.SemaphoreType.DMA,
            pltpu.SemaphoreType.DMA,
            pltpu.SemaphoreType.DMA,
        ],
    )


def _edge(q, k, v, src, dst, zn, zd):
    return _make_edge()(q, k, v, src, dst, zn, zd)


def kernel(x, edge_index, W1q, b1q, W1k, b1k, W1v, b1v, W1s, b1s,
           W2q, b2q, W2k, b2k, W2v, b2v, W2s, b2s):
    src = edge_index[0]
    dst = edge_index[1]
    w1 = jnp.concatenate([W1q.T, W1k.T, W1v.T, W1s.T], axis=1)
    b1 = jnp.concatenate([b1q, b1k, b1v, b1s])[None, :]
    w2 = jnp.concatenate([W2q.T, W2k.T, W2v.T, W2s.T], axis=1)
    b2 = jnp.concatenate([b2q, b2k, b2v, b2s])[None, :]
    xp = jnp.zeros((_NP, _D), jnp.float32).at[:_N].set(x)
    zn = jnp.zeros((_RPT, _D), jnp.float32)
    zd = jnp.zeros((_NP,), jnp.float32)
    ones = jnp.ones((_NS, 1), jnp.float32)

    q1, k1, v1, s1 = _proj(xp, w1, b1)
    p1, dp1 = _edge(q1, k1, v1, src, dst, zn, zd)
    q2, k2, v2, s2 = _comb_proj(p1, dp1, s1, ones, w2, b2)
    p2, dp2 = _edge(q2, k2, v2, src, dst, zn, zd)
    return _comb(p2, dp2, s2, ones)[:_N]


# speedup vs baseline: 12.3284x; 1.4498x over previous
import functools

import jax
import jax.numpy as jnp
from jax import lax
from jax.experimental import pallas as pl
from jax.experimental.pallas import tpu as pltpu
from jax.experimental.pallas import tpu_sc as plsc

_N = 10000
_E = 320000
_D = 128
_NC = 2
_NS = 16
_NW = _NC * _NS
_EPW = _E // _NW
_C = 40
_NG = _C // 16
_NCHUNK = _EPW // _C
_NP = 10240
_RPT = _NP // _NS
_INV_SQRT_D = 1.0 / (_D ** 0.5)

_BN = 640
_GRID = _NP // _BN


def _proj_body(x_ref, w_ref, b_ref, q_ref, k_ref, v_ref, s_ref):
    acc = jnp.dot(x_ref[...], w_ref[...],
                  preferred_element_type=jnp.float32) + b_ref[...]
    q_ref[...] = acc[:, :_D]
    k_ref[...] = acc[:, _D:2 * _D]
    v_ref[...] = acc[:, 2 * _D:3 * _D]
    s_ref[...] = acc[:, 3 * _D:]


def _qkvs_specs():
    return dict(
        out_specs=[
            pl.BlockSpec((_BN, _D), lambda i: (i, 0)),
            pl.BlockSpec((_BN, _D), lambda i: (i, 0)),
            pl.BlockSpec((_BN, _D), lambda i: (i, 0)),
            pl.BlockSpec((_BN, _D), lambda i: (i, 0)),
        ],
        out_shape=[
            jax.ShapeDtypeStruct((_NP, _D), jnp.float32),
            jax.ShapeDtypeStruct((_NP, _D), jnp.float32),
            jax.ShapeDtypeStruct((_NP, _D), jnp.float32),
            jax.ShapeDtypeStruct((_NP, _D), jnp.float32),
        ],
    )


def _proj(x, w, b):
    return pl.pallas_call(
        _proj_body,
        grid=(_GRID,),
        in_specs=[
            pl.BlockSpec((_BN, _D), lambda i: (i, 0)),
            pl.BlockSpec((_D, 4 * _D), lambda i: (0, 0)),
            pl.BlockSpec((1, 4 * _D), lambda i: (0, 0)),
        ],
        **_qkvs_specs(),
    )(x, w, b)


def _combine_h(p_ref, dp_ref, s_ref, ones_ref):
    num = p_ref[0] + p_ref[1]
    dsum = dp_ref[0] + dp_ref[1]
    den = lax.dot_general(dsum, ones_ref[...],
                          (((0,), (0,)), ((), ())),
                          preferred_element_type=jnp.float32)
    return num / (den + 1e-16) + s_ref[...]


def _comb_proj_body(p_ref, dp_ref, s1_ref, ones_ref, w_ref, b_ref,
                    q_ref, k_ref, v_ref, s_ref):
    h = jnp.maximum(_combine_h(p_ref, dp_ref, s1_ref, ones_ref), 0.0)
    acc = jnp.dot(h, w_ref[...],
                  preferred_element_type=jnp.float32) + b_ref[...]
    q_ref[...] = acc[:, :_D]
    k_ref[...] = acc[:, _D:2 * _D]
    v_ref[...] = acc[:, 2 * _D:3 * _D]
    s_ref[...] = acc[:, 3 * _D:]


def _comb_proj(p, dp, s1, ones, w, b):
    return pl.pallas_call(
        _comb_proj_body,
        grid=(_GRID,),
        in_specs=[
            pl.BlockSpec((2, _BN, _D), lambda i: (0, i, 0)),
            pl.BlockSpec((2, _NS, _BN), lambda i: (0, 0, i)),
            pl.BlockSpec((_BN, _D), lambda i: (i, 0)),
            pl.BlockSpec((_NS, 1), lambda i: (0, 0)),
            pl.BlockSpec((_D, 4 * _D), lambda i: (0, 0)),
            pl.BlockSpec((1, 4 * _D), lambda i: (0, 0)),
        ],
        **_qkvs_specs(),
    )(p, dp, s1, ones, w, b)


def _comb_body(p_ref, dp_ref, s2_ref, ones_ref, o_ref):
    o_ref[...] = _combine_h(p_ref, dp_ref, s2_ref, ones_ref)


def _comb(p, dp, s2, ones):
    return pl.pallas_call(
        _comb_body,
        grid=(_GRID,),
        in_specs=[
            pl.BlockSpec((2, _BN, _D), lambda i: (0, i, 0)),
            pl.BlockSpec((2, _NS, _BN), lambda i: (0, 0, i)),
            pl.BlockSpec((_BN, _D), lambda i: (i, 0)),
            pl.BlockSpec((_NS, 1), lambda i: (0, 0)),
        ],
        out_specs=pl.BlockSpec((_BN, _D), lambda i: (i, 0)),
        out_shape=jax.ShapeDtypeStruct((_NP, _D), jnp.float32),
    )(p, dp, s2, ones)


_CPT = 250


def _edge_body(q_hbm, k_hbm, v_hbm, src_hbm, dst_hbm, zn_hbm, zd_hbm,
               part_hbm, dpart_hbm,
               srcv0, srcv1, dstv0, dstv1, dstc0, dstc1,
               qb0, qb1, kb0, kb1, vb0, vb1, ebuf, evec, dtile,
               acc, semg0, semg1, semi, sems):
    srcv = (srcv0, srcv1)
    dstv = (dstv0, dstv1)
    dstc = (dstc0, dstc1)
    qb = (qb0, qb1)
    kb = (kb0, kb1)
    vb = (vb0, vb1)
    semg = (semg0, semg1)

    cid = lax.axis_index("c")
    sid = lax.axis_index("s")
    wid = sid * _NC + cid

    pltpu.sync_copy(zn_hbm, acc.at[pl.ds(sid * _RPT, _RPT)])
    pltpu.sync_copy(zd_hbm, dtile)
    plsc.subcore_barrier()

    lanes = lax.iota(jnp.int32, 16)
    base_w = wid * _EPW

    def idx_fetch(c, s, sync):
        base = base_w + c * _C
        if sync:
            pltpu.sync_copy(src_hbm.at[pl.ds(base, _C)],
                            srcv[s].at[pl.ds(0, _C)])
            pltpu.sync_copy(dst_hbm.at[pl.ds(base, _C)],
                            dstv[s].at[pl.ds(0, _C)])
        else:
            pltpu.async_copy(src_hbm.at[pl.ds(base, _C)],
                             srcv[s].at[pl.ds(0, _C)], semi)
            pltpu.async_copy(dst_hbm.at[pl.ds(base, _C)],
                             dstv[s].at[pl.ds(0, _C)], semi)

    def idx_wait(s):
        pltpu.make_async_copy(src_hbm.at[pl.ds(0, _C)],
                              srcv[s].at[pl.ds(0, _C)], semi).wait()
        pltpu.make_async_copy(dst_hbm.at[pl.ds(0, _C)],
                              dstv[s].at[pl.ds(0, _C)], semi).wait()

    def gathers_issue(s):
        pltpu.async_copy(k_hbm.at[srcv[s].at[pl.ds(0, _C)]], kb[s], semg[s])
        pltpu.async_copy(v_hbm.at[srcv[s].at[pl.ds(0, _C)]], vb[s], semg[s])
        pltpu.async_copy(q_hbm.at[dstv[s].at[pl.ds(0, _C)]], qb[s], semg[s])

    def gathers_wait(s):
        for ref in (kb[s], vb[s], qb[s]):
            pltpu.make_async_copy(
                k_hbm.at[srcv[s].at[pl.ds(0, _C)]], ref, semg[s]).wait()

    def scatter_issue(s):
        return pltpu.async_copy(vb[s], acc.at[dstc[s]], sems, add=True)

    def scatter_wait(s):
        pltpu.make_async_copy(vb[s], acc.at[dstc[s]], sems).wait()

    def compute(s, dsts):
        qbuf, kbuf, vbuf = qb[s], kb[s], vb[s]
        for gi, (off, sz) in enumerate(((0, 16), (16, 16), (32, 8))):
            def p1(ei, carry3):
                e = off + ei
                p0 = qbuf[e, pl.ds(0, 16)] * kbuf[e, pl.ds(0, 16)]
                p1_ = qbuf[e, pl.ds(16, 16)] * kbuf[e, pl.ds(16, 16)]
                p2 = qbuf[e, pl.ds(32, 16)] * kbuf[e, pl.ds(32, 16)]
                p3_ = qbuf[e, pl.ds(48, 16)] * kbuf[e, pl.ds(48, 16)]
                p4 = qbuf[e, pl.ds(64, 16)] * kbuf[e, pl.ds(64, 16)]
                p5 = qbuf[e, pl.ds(80, 16)] * kbuf[e, pl.ds(80, 16)]
                p6 = qbuf[e, pl.ds(96, 16)] * kbuf[e, pl.ds(96, 16)]
                p7 = qbuf[e, pl.ds(112, 16)] * kbuf[e, pl.ds(112, 16)]
                ebuf[ei, pl.ds(0, 16)] = (
                    ((p0 + p1_) + (p2 + p3_)) + ((p4 + p5) + (p6 + p7)))
                return carry3

            lax.fori_loop(0, sz, p1, 0, unroll=4)

            cols = [plsc.load_gather(
                ebuf, [lanes, jnp.full((16,), l, jnp.int32)])
                for l in range(16)]
            t0 = ((cols[0] + cols[1]) + (cols[2] + cols[3]))
            t1 = ((cols[4] + cols[5]) + (cols[6] + cols[7]))
            t2 = ((cols[8] + cols[9]) + (cols[10] + cols[11]))
            t3 = ((cols[12] + cols[13]) + (cols[14] + cols[15]))
            exv = jnp.exp(((t0 + t1) + (t2 + t3)) * _INV_SQRT_D)
            evec[pl.ds(0, 16)] = exv
            dst16 = dsts[gi]
            if sz == 16:
                plsc.addupdate_scatter(dtile, [dst16], exv)
            else:
                plsc.addupdate_scatter(dtile, [dst16], exv, mask=lanes < sz)

            def p3(ei, carry3):
                e = off + ei
                exb = plsc.load_gather(
                    evec, [jnp.full((16,), ei, jnp.int32)])
                for j in range(8):
                    vbuf[e, pl.ds(16 * j, 16)] = (
                        vbuf[e, pl.ds(16 * j, 16)] * exb)
                return carry3

            lax.fori_loop(0, sz, p3, 0, unroll=4)

    def half(i, b):
        if b == 0:
            with_s1 = lambda f: pl.when(i >= 1)(f)
            with_s23 = lambda f: f()
            with_s6 = lambda f: pl.when(i < 124)(f)
        else:
            with_s1 = lambda f: f()
            with_s23 = lambda f: pl.when(i < 124)(f)
            with_s6 = lambda f: pl.when(i < 124)(f)
        c = 2 * i + b
        with_s1(lambda: scatter_wait(1 - b))
        with_s23(lambda: idx_wait(1 - b))
        with_s23(lambda: gathers_issue(1 - b))
        gathers_wait(b)
        d0 = dstv[b][pl.ds(0, 16)]
        d1 = dstv[b][pl.ds(16, 16)]
        d2 = dstv[b][pl.ds(32, 16)]
        dstc[b][pl.ds(0, 16)] = d0
        dstc[b][pl.ds(16, 16)] = d1
        plsc.store_scatter(dstc[b], [lanes + 32], d2, mask=lanes < 8)
        with_s6(lambda: idx_fetch(c + 2, b, sync=False))
        compute(b, (d0, d1, d2))
        scatter_issue(b)

    idx_fetch(0, 0, sync=True)
    gathers_issue(0)
    idx_fetch(1, 1, sync=False)

    def step(i, carry):
        half(i, 0)
        half(i, 1)
        return carry

    lax.fori_loop(0, _CPT // 2, step, 0)
    scatter_wait(1)

    plsc.subcore_barrier()
    pltpu.sync_copy(acc.at[pl.ds(sid * _RPT, _RPT)],
                    part_hbm.at[cid, pl.ds(sid * _RPT, _RPT)])
    pltpu.sync_copy(dtile, dpart_hbm.at[cid, sid])


@functools.cache
def _make_edge():
    mesh = plsc.VectorSubcoreMesh(
        core_axis_name="c", subcore_axis_name="s",
        num_cores=_NC, num_subcores=_NS)
    return pl.kernel(
        _edge_body,
        out_type=(jax.ShapeDtypeStruct((_NC, _NP, _D), jnp.float32),
                  jax.ShapeDtypeStruct((_NC, _NS, _NP), jnp.float32)),
        mesh=mesh,
        compiler_params=pltpu.CompilerParams(needs_layout_passes=False),
        scratch_types=[
            pltpu.VMEM((48,), jnp.int32),
            pltpu.VMEM((48,), jnp.int32),
            pltpu.VMEM((48,), jnp.int32),
            pltpu.VMEM((48,), jnp.int32),
            pltpu.VMEM((_C,), jnp.int32),
            pltpu.VMEM((_C,), jnp.int32),
            pltpu.VMEM((_C, _D), jnp.float32),
            pltpu.VMEM((_C, _D), jnp.float32),
            pltpu.VMEM((_C, _D), jnp.float32),
            pltpu.VMEM((_C, _D), jnp.float32),
            pltpu.VMEM((_C, _D), jnp.float32),
            pltpu.VMEM((_C, _D), jnp.float32),
            pltpu.VMEM((16, 16), jnp.float32),
            pltpu.VMEM((16,), jnp.float32),
            pltpu.VMEM((_NP,), jnp.float32),
            pltpu.VMEM_SHARED((_NP, _D), jnp.float32),
            pltpu.SemaphoreType.DMA,
            pltpu.SemaphoreType.DMA,
            pltpu.SemaphoreType.DMA,
            pltpu.SemaphoreType.DMA,
        ],
    )


def _edge(q, k, v, src, dst, zn, zd):
    return _make_edge()(q, k, v, src, dst, zn, zd)


def kernel(x, edge_index, W1q, b1q, W1k, b1k, W1v, b1v, W1s, b1s,
           W2q, b2q, W2k, b2k, W2v, b2v, W2s, b2s):
    src = edge_index[0]
    dst = edge_index[1]
    w1 = jnp.concatenate([W1q.T, W1k.T, W1v.T, W1s.T], axis=1)
    b1 = jnp.concatenate([b1q, b1k, b1v, b1s])[None, :]
    w2 = jnp.concatenate([W2q.T, W2k.T, W2v.T, W2s.T], axis=1)
    b2 = jnp.concatenate([b2q, b2k, b2v, b2s])[None, :]
    xp = jnp.zeros((_NP, _D), jnp.float32).at[:_N].set(x)
    zn = jnp.zeros((_RPT, _D), jnp.float32)
    zd = jnp.zeros((_NP,), jnp.float32)
    ones = jnp.ones((_NS, 1), jnp.float32)

    q1, k1, v1, s1 = _proj(xp, w1, b1)
    p1, dp1 = _edge(q1, k1, v1, src, dst, zn, zd)
    q2, k2, v2, s2 = _comb_proj(p1, dp1, s1, ones, w2, b2)
    p2, dp2 = _edge(q2, k2, v2, src, dst, zn, zd)
    return _comb(p2, dp2, s2, ones)[:_N]


# speedup vs baseline: 12.6428x; 1.0255x over previous
import functools

import jax
import jax.numpy as jnp
from jax import lax
from jax.experimental import pallas as pl
from jax.experimental.pallas import tpu as pltpu
from jax.experimental.pallas import tpu_sc as plsc

_N = 10000
_E = 320000
_D = 128
_NC = 2
_NS = 16
_NW = _NC * _NS
_EPW = _E // _NW
_C = 40
_NG = _C // 16
_NCHUNK = _EPW // _C
_NP = 10240
_RPT = _NP // _NS
_INV_SQRT_D = 1.0 / (_D ** 0.5)

_BN = 640
_GRID = _NP // _BN


def _proj_body(x_ref, w_ref, b_ref, q_ref, k_ref, v_ref, s_ref):
    acc = jnp.dot(x_ref[...], w_ref[...],
                  preferred_element_type=jnp.float32) + b_ref[...]
    q_ref[...] = acc[:, :_D]
    k_ref[...] = acc[:, _D:2 * _D]
    v_ref[...] = acc[:, 2 * _D:3 * _D]
    s_ref[...] = acc[:, 3 * _D:]


def _qkvs_specs():
    return dict(
        out_specs=[
            pl.BlockSpec((_BN, _D), lambda i: (i, 0)),
            pl.BlockSpec((_BN, _D), lambda i: (i, 0)),
            pl.BlockSpec((_BN, _D), lambda i: (i, 0)),
            pl.BlockSpec((_BN, _D), lambda i: (i, 0)),
        ],
        out_shape=[
            jax.ShapeDtypeStruct((_NP, _D), jnp.float32),
            jax.ShapeDtypeStruct((_NP, _D), jnp.float32),
            jax.ShapeDtypeStruct((_NP, _D), jnp.float32),
            jax.ShapeDtypeStruct((_NP, _D), jnp.float32),
        ],
    )


def _proj(x, w, b):
    return pl.pallas_call(
        _proj_body,
        grid=(_GRID,),
        in_specs=[
            pl.BlockSpec((_BN, _D), lambda i: (i, 0)),
            pl.BlockSpec((_D, 4 * _D), lambda i: (0, 0)),
            pl.BlockSpec((1, 4 * _D), lambda i: (0, 0)),
        ],
        **_qkvs_specs(),
    )(x, w, b)


def _combine_h(p_ref, dp_ref, s_ref, ones_ref):
    num = p_ref[0] + p_ref[1]
    dsum = dp_ref[0] + dp_ref[1]
    den = lax.dot_general(dsum, ones_ref[...],
                          (((0,), (0,)), ((), ())),
                          preferred_element_type=jnp.float32)
    return num / (den + 1e-16) + s_ref[...]


def _comb_proj_body(p_ref, dp_ref, s1_ref, ones_ref, w_ref, b_ref,
                    q_ref, k_ref, v_ref, s_ref):
    h = jnp.maximum(_combine_h(p_ref, dp_ref, s1_ref, ones_ref), 0.0)
    acc = jnp.dot(h, w_ref[...],
                  preferred_element_type=jnp.float32) + b_ref[...]
    q_ref[...] = acc[:, :_D]
    k_ref[...] = acc[:, _D:2 * _D]
    v_ref[...] = acc[:, 2 * _D:3 * _D]
    s_ref[...] = acc[:, 3 * _D:]


def _comb_proj(p, dp, s1, ones, w, b):
    return pl.pallas_call(
        _comb_proj_body,
        grid=(_GRID,),
        in_specs=[
            pl.BlockSpec((2, _BN, _D), lambda i: (0, i, 0)),
            pl.BlockSpec((2, _NS, _BN), lambda i: (0, 0, i)),
            pl.BlockSpec((_BN, _D), lambda i: (i, 0)),
            pl.BlockSpec((_NS, 1), lambda i: (0, 0)),
            pl.BlockSpec((_D, 4 * _D), lambda i: (0, 0)),
            pl.BlockSpec((1, 4 * _D), lambda i: (0, 0)),
        ],
        **_qkvs_specs(),
    )(p, dp, s1, ones, w, b)


def _comb_body(p_ref, dp_ref, s2_ref, ones_ref, o_ref):
    o_ref[...] = _combine_h(p_ref, dp_ref, s2_ref, ones_ref)


def _comb(p, dp, s2, ones):
    return pl.pallas_call(
        _comb_body,
        grid=(_GRID,),
        in_specs=[
            pl.BlockSpec((2, _BN, _D), lambda i: (0, i, 0)),
            pl.BlockSpec((2, _NS, _BN), lambda i: (0, 0, i)),
            pl.BlockSpec((_BN, _D), lambda i: (i, 0)),
            pl.BlockSpec((_NS, 1), lambda i: (0, 0)),
        ],
        out_specs=pl.BlockSpec((_BN, _D), lambda i: (i, 0)),
        out_shape=jax.ShapeDtypeStruct((_NP, _D), jnp.float32),
    )(p, dp, s2, ones)


_CPT = 250


def _edge_body(q_hbm, k_hbm, v_hbm, src_hbm, dst_hbm, zn_hbm, zd_hbm,
               part_hbm, dpart_hbm,
               srcv0, srcv1, dstv0, dstv1, dstc0, dstc1,
               qb0, qb1, kb0, kb1, vb0, vb1, ebuf, evec, dtile,
               acc, semg0, semg1, semi, sems):
    srcv = (srcv0, srcv1)
    dstv = (dstv0, dstv1)
    dstc = (dstc0, dstc1)
    qb = (qb0, qb1)
    kb = (kb0, kb1)
    vb = (vb0, vb1)
    semg = (semg0, semg1)

    cid = lax.axis_index("c")
    sid = lax.axis_index("s")
    wid = sid * _NC + cid

    pltpu.sync_copy(zn_hbm, acc.at[pl.ds(sid * _RPT, _RPT)])
    pltpu.sync_copy(zd_hbm, dtile)
    plsc.subcore_barrier()

    lanes = lax.iota(jnp.int32, 16)
    base_w = wid * _EPW

    def idx_fetch(c, s, sync):
        base = base_w + c * _C
        if sync:
            pltpu.sync_copy(src_hbm.at[pl.ds(base, _C)],
                            srcv[s].at[pl.ds(0, _C)])
            pltpu.sync_copy(dst_hbm.at[pl.ds(base, _C)],
                            dstv[s].at[pl.ds(0, _C)])
        else:
            pltpu.async_copy(src_hbm.at[pl.ds(base, _C)],
                             srcv[s].at[pl.ds(0, _C)], semi)
            pltpu.async_copy(dst_hbm.at[pl.ds(base, _C)],
                             dstv[s].at[pl.ds(0, _C)], semi)

    def idx_wait(s):
        pltpu.make_async_copy(src_hbm.at[pl.ds(0, _C)],
                              srcv[s].at[pl.ds(0, _C)], semi).wait()
        pltpu.make_async_copy(dst_hbm.at[pl.ds(0, _C)],
                              dstv[s].at[pl.ds(0, _C)], semi).wait()

    def gathers_issue(s):
        pltpu.async_copy(k_hbm.at[srcv[s].at[pl.ds(0, _C)]], kb[s], semg[s])
        pltpu.async_copy(v_hbm.at[srcv[s].at[pl.ds(0, _C)]], vb[s], semg[s])
        pltpu.async_copy(q_hbm.at[dstv[s].at[pl.ds(0, _C)]], qb[s], semg[s])

    def gathers_wait(s):
        for ref in (kb[s], vb[s], qb[s]):
            pltpu.make_async_copy(
                k_hbm.at[srcv[s].at[pl.ds(0, _C)]], ref, semg[s]).wait()

    def scatter_issue(s):
        return pltpu.async_copy(vb[s], acc.at[dstc[s]], sems, add=True)

    def scatter_wait(s):
        pltpu.make_async_copy(vb[s], acc.at[dstc[s]], sems).wait()

    def compute(s, dsts):
        qbuf, kbuf, vbuf = qb[s], kb[s], vb[s]
        for gi, (off, sz) in enumerate(((0, 16), (16, 16), (32, 8))):
            def p1(ei, carry3):
                e = off + ei
                p0 = qbuf[e, pl.ds(0, 16)] * kbuf[e, pl.ds(0, 16)]
                p1_ = qbuf[e, pl.ds(16, 16)] * kbuf[e, pl.ds(16, 16)]
                p2 = qbuf[e, pl.ds(32, 16)] * kbuf[e, pl.ds(32, 16)]
                p3_ = qbuf[e, pl.ds(48, 16)] * kbuf[e, pl.ds(48, 16)]
                p4 = qbuf[e, pl.ds(64, 16)] * kbuf[e, pl.ds(64, 16)]
                p5 = qbuf[e, pl.ds(80, 16)] * kbuf[e, pl.ds(80, 16)]
                p6 = qbuf[e, pl.ds(96, 16)] * kbuf[e, pl.ds(96, 16)]
                p7 = qbuf[e, pl.ds(112, 16)] * kbuf[e, pl.ds(112, 16)]
                ebuf[ei, pl.ds(0, 16)] = (
                    ((p0 + p1_) + (p2 + p3_)) + ((p4 + p5) + (p6 + p7)))
                return carry3

            lax.fori_loop(0, sz, p1, 0, unroll=8)

            cols = [plsc.load_gather(
                ebuf, [lanes, jnp.full((16,), l, jnp.int32)])
                for l in range(16)]
            t0 = ((cols[0] + cols[1]) + (cols[2] + cols[3]))
            t1 = ((cols[4] + cols[5]) + (cols[6] + cols[7]))
            t2 = ((cols[8] + cols[9]) + (cols[10] + cols[11]))
            t3 = ((cols[12] + cols[13]) + (cols[14] + cols[15]))
            exv = jnp.exp(((t0 + t1) + (t2 + t3)) * _INV_SQRT_D)
            evec[pl.ds(0, 16)] = exv
            dst16 = dsts[gi]
            if sz == 16:
                plsc.addupdate_scatter(dtile, [dst16], exv)
            else:
                plsc.addupdate_scatter(dtile, [dst16], exv, mask=lanes < sz)

            def p3(ei, carry3):
                e = off + ei
                exb = plsc.load_gather(
                    evec, [jnp.full((16,), ei, jnp.int32)])
                for j in range(8):
                    vbuf[e, pl.ds(16 * j, 16)] = (
                        vbuf[e, pl.ds(16 * j, 16)] * exb)
                return carry3

            lax.fori_loop(0, sz, p3, 0, unroll=8)

    def half(i, b):
        if b == 0:
            with_s1 = lambda f: pl.when(i >= 1)(f)
            with_s23 = lambda f: f()
            with_s6 = lambda f: pl.when(i < 124)(f)
        else:
            with_s1 = lambda f: f()
            with_s23 = lambda f: pl.when(i < 124)(f)
            with_s6 = lambda f: pl.when(i < 124)(f)
        c = 2 * i + b
        with_s1(lambda: scatter_wait(1 - b))
        with_s23(lambda: idx_wait(1 - b))
        with_s23(lambda: gathers_issue(1 - b))
        gathers_wait(b)
        d0 = dstv[b][pl.ds(0, 16)]
        d1 = dstv[b][pl.ds(16, 16)]
        d2 = dstv[b][pl.ds(32, 16)]
        dstc[b][pl.ds(0, 16)] = d0
        dstc[b][pl.ds(16, 16)] = d1
        plsc.store_scatter(dstc[b], [lanes + 32], d2, mask=lanes < 8)
        with_s6(lambda: idx_fetch(c + 2, b, sync=False))
        compute(b, (d0, d1, d2))
        scatter_issue(b)

    idx_fetch(0, 0, sync=True)
    gathers_issue(0)
    idx_fetch(1, 1, sync=False)

    def step(i, carry):
        half(i, 0)
        half(i, 1)
        return carry

    lax.fori_loop(0, _CPT // 2, step, 0)
    scatter_wait(1)

    plsc.subcore_barrier()
    pltpu.sync_copy(acc.at[pl.ds(sid * _RPT, _RPT)],
                    part_hbm.at[cid, pl.ds(sid * _RPT, _RPT)])
    pltpu.sync_copy(dtile, dpart_hbm.at[cid, sid])


@functools.cache
def _make_edge():
    mesh = plsc.VectorSubcoreMesh(
        core_axis_name="c", subcore_axis_name="s",
        num_cores=_NC, num_subcores=_NS)
    return pl.kernel(
        _edge_body,
        out_type=(jax.ShapeDtypeStruct((_NC, _NP, _D), jnp.float32),
                  jax.ShapeDtypeStruct((_NC, _NS, _NP), jnp.float32)),
        mesh=mesh,
        compiler_params=pltpu.CompilerParams(needs_layout_passes=False),
        scratch_types=[
            pltpu.VMEM((48,), jnp.int32),
            pltpu.VMEM((48,), jnp.int32),
            pltpu.VMEM((48,), jnp.int32),
            pltpu.VMEM((48,), jnp.int32),
            pltpu.VMEM((_C,), jnp.int32),
            pltpu.VMEM((_C,), jnp.int32),
            pltpu.VMEM((_C, _D), jnp.float32),
            pltpu.VMEM((_C, _D), jnp.float32),
            pltpu.VMEM((_C, _D), jnp.float32),
            pltpu.VMEM((_C, _D), jnp.float32),
            pltpu.VMEM((_C, _D), jnp.float32),
            pltpu.VMEM((_C, _D), jnp.float32),
            pltpu.VMEM((16, 16), jnp.float32),
            pltpu.VMEM((16,), jnp.float32),
            pltpu.VMEM((_NP,), jnp.float32),
            pltpu.VMEM_SHARED((_NP, _D), jnp.float32),
            pltpu.SemaphoreType.DMA,
            pltpu.SemaphoreType.DMA,
            pltpu.SemaphoreType.DMA,
            pltpu.SemaphoreType.DMA,
        ],
    )


def _edge(q, k, v, src, dst, zn, zd):
    return _make_edge()(q, k, v, src, dst, zn, zd)


def kernel(x, edge_index, W1q, b1q, W1k, b1k, W1v, b1v, W1s, b1s,
           W2q, b2q, W2k, b2k, W2v, b2v, W2s, b2s):
    src = edge_index[0]
    dst = edge_index[1]
    w1 = jnp.concatenate([W1q.T, W1k.T, W1v.T, W1s.T], axis=1)
    b1 = jnp.concatenate([b1q, b1k, b1v, b1s])[None, :]
    w2 = jnp.concatenate([W2q.T, W2k.T, W2v.T, W2s.T], axis=1)
    b2 = jnp.concatenate([b2q, b2k, b2v, b2s])[None, :]
    xp = jnp.zeros((_NP, _D), jnp.float32).at[:_N].set(x)
    zn = jnp.zeros((_RPT, _D), jnp.float32)
    zd = jnp.zeros((_NP,), jnp.float32)
    ones = jnp.ones((_NS, 1), jnp.float32)

    q1, k1, v1, s1 = _proj(xp, w1, b1)
    p1, dp1 = _edge(q1, k1, v1, src, dst, zn, zd)
    q2, k2, v2, s2 = _comb_proj(p1, dp1, s1, ones, w2, b2)
    p2, dp2 = _edge(q2, k2, v2, src, dst, zn, zd)
    return _comb(p2, dp2, s2, ones)[:_N]


# speedup vs baseline: 12.9384x; 1.0234x over previous
import functools

import jax
import jax.numpy as jnp
from jax import lax
from jax.experimental import pallas as pl
from jax.experimental.pallas import tpu as pltpu
from jax.experimental.pallas import tpu_sc as plsc

_N = 10000
_E = 320000
_D = 128
_NC = 2
_NS = 16
_NW = _NC * _NS
_EPW = _E // _NW
_C = 40
_NG = _C // 16
_NCHUNK = _EPW // _C
_NP = 10240
_RPT = _NP // _NS
_INV_SQRT_D = 1.0 / (_D ** 0.5)

_BN = 640
_GRID = _NP // _BN


def _proj_body(x_ref, w_ref, b_ref, q_ref, k_ref, v_ref, s_ref):
    acc = jnp.dot(x_ref[...], w_ref[...],
                  preferred_element_type=jnp.float32) + b_ref[...]
    q_ref[...] = acc[:, :_D]
    k_ref[...] = acc[:, _D:2 * _D]
    v_ref[...] = acc[:, 2 * _D:3 * _D]
    s_ref[...] = acc[:, 3 * _D:]


def _qkvs_specs():
    return dict(
        out_specs=[
            pl.BlockSpec((_BN, _D), lambda i: (i, 0)),
            pl.BlockSpec((_BN, _D), lambda i: (i, 0)),
            pl.BlockSpec((_BN, _D), lambda i: (i, 0)),
            pl.BlockSpec((_BN, _D), lambda i: (i, 0)),
        ],
        out_shape=[
            jax.ShapeDtypeStruct((_NP, _D), jnp.float32),
            jax.ShapeDtypeStruct((_NP, _D), jnp.float32),
            jax.ShapeDtypeStruct((_NP, _D), jnp.float32),
            jax.ShapeDtypeStruct((_NP, _D), jnp.float32),
        ],
    )


def _proj(x, w, b):
    return pl.pallas_call(
        _proj_body,
        grid=(_GRID,),
        in_specs=[
            pl.BlockSpec((_BN, _D), lambda i: (i, 0)),
            pl.BlockSpec((_D, 4 * _D), lambda i: (0, 0)),
            pl.BlockSpec((1, 4 * _D), lambda i: (0, 0)),
        ],
        **_qkvs_specs(),
    )(x, w, b)


def _combine_h(p_ref, dp_ref, s_ref, ones_ref):
    num = p_ref[0] + p_ref[1]
    dsum = dp_ref[0] + dp_ref[1]
    den = lax.dot_general(dsum, ones_ref[...],
                          (((0,), (0,)), ((), ())),
                          preferred_element_type=jnp.float32)
    return num / (den + 1e-16) + s_ref[...]


def _comb_proj_body(p_ref, dp_ref, s1_ref, ones_ref, w_ref, b_ref,
                    q_ref, k_ref, v_ref, s_ref):
    h = jnp.maximum(_combine_h(p_ref, dp_ref, s1_ref, ones_ref), 0.0)
    acc = jnp.dot(h, w_ref[...],
                  preferred_element_type=jnp.float32) + b_ref[...]
    q_ref[...] = acc[:, :_D]
    k_ref[...] = acc[:, _D:2 * _D]
    v_ref[...] = acc[:, 2 * _D:3 * _D]
    s_ref[...] = acc[:, 3 * _D:]


def _comb_proj(p, dp, s1, ones, w, b):
    return pl.pallas_call(
        _comb_proj_body,
        grid=(_GRID,),
        in_specs=[
            pl.BlockSpec((2, _BN, _D), lambda i: (0, i, 0)),
            pl.BlockSpec((2, _NS, _BN), lambda i: (0, 0, i)),
            pl.BlockSpec((_BN, _D), lambda i: (i, 0)),
            pl.BlockSpec((_NS, 1), lambda i: (0, 0)),
            pl.BlockSpec((_D, 4 * _D), lambda i: (0, 0)),
            pl.BlockSpec((1, 4 * _D), lambda i: (0, 0)),
        ],
        **_qkvs_specs(),
    )(p, dp, s1, ones, w, b)


def _comb_body(p_ref, dp_ref, s2_ref, ones_ref, o_ref):
    o_ref[...] = _combine_h(p_ref, dp_ref, s2_ref, ones_ref)


def _comb(p, dp, s2, ones):
    return pl.pallas_call(
        _comb_body,
        grid=(_GRID,),
        in_specs=[
            pl.BlockSpec((2, _BN, _D), lambda i: (0, i, 0)),
            pl.BlockSpec((2, _NS, _BN), lambda i: (0, 0, i)),
            pl.BlockSpec((_BN, _D), lambda i: (i, 0)),
            pl.BlockSpec((_NS, 1), lambda i: (0, 0)),
        ],
        out_specs=pl.BlockSpec((_BN, _D), lambda i: (i, 0)),
        out_shape=jax.ShapeDtypeStruct((_NP, _D), jnp.float32),
    )(p, dp, s2, ones)


_CPT = 250


def _edge_body(q_hbm, k_hbm, v_hbm, src_hbm, dst_hbm, zn_hbm, zd_hbm,
               part_hbm, dpart_hbm,
               srcv0, srcv1, dstv0, dstv1, dstc0, dstc1,
               qb0, qb1, kb0, kb1, vb0, vb1, ebuf, evec, dtile,
               acc, semg0, semg1, semi, sems):
    srcv = (srcv0, srcv1)
    dstv = (dstv0, dstv1)
    dstc = (dstc0, dstc1)
    qb = (qb0, qb1)
    kb = (kb0, kb1)
    vb = (vb0, vb1)
    semg = (semg0, semg1)

    cid = lax.axis_index("c")
    sid = lax.axis_index("s")
    wid = sid * _NC + cid

    pltpu.sync_copy(zn_hbm, acc.at[pl.ds(sid * _RPT, _RPT)])
    pltpu.sync_copy(zd_hbm, dtile)
    plsc.subcore_barrier()

    lanes = lax.iota(jnp.int32, 16)
    base_w = wid * _EPW

    def idx_fetch(c, s, sync):
        base = base_w + c * _C
        if sync:
            pltpu.sync_copy(src_hbm.at[pl.ds(base, _C)],
                            srcv[s].at[pl.ds(0, _C)])
            pltpu.sync_copy(dst_hbm.at[pl.ds(base, _C)],
                            dstv[s].at[pl.ds(0, _C)])
        else:
            pltpu.async_copy(src_hbm.at[pl.ds(base, _C)],
                             srcv[s].at[pl.ds(0, _C)], semi)
            pltpu.async_copy(dst_hbm.at[pl.ds(base, _C)],
                             dstv[s].at[pl.ds(0, _C)], semi)

    def idx_wait(s):
        pltpu.make_async_copy(src_hbm.at[pl.ds(0, _C)],
                              srcv[s].at[pl.ds(0, _C)], semi).wait()
        pltpu.make_async_copy(dst_hbm.at[pl.ds(0, _C)],
                              dstv[s].at[pl.ds(0, _C)], semi).wait()

    def gathers_issue(s):
        pltpu.async_copy(k_hbm.at[srcv[s].at[pl.ds(0, _C)]], kb[s], semg[s])
        pltpu.async_copy(v_hbm.at[srcv[s].at[pl.ds(0, _C)]], vb[s], semg[s])
        pltpu.async_copy(q_hbm.at[dstv[s].at[pl.ds(0, _C)]], qb[s], semg[s])

    def gathers_wait(s):
        for ref in (kb[s], vb[s], qb[s]):
            pltpu.make_async_copy(
                k_hbm.at[srcv[s].at[pl.ds(0, _C)]], ref, semg[s]).wait()

    def scatter_issue(s):
        return pltpu.async_copy(vb[s], acc.at[dstc[s]], sems, add=True)

    def scatter_wait(s):
        pltpu.make_async_copy(vb[s], acc.at[dstc[s]], sems).wait()

    def compute(s, dsts):
        qbuf, kbuf, vbuf = qb[s], kb[s], vb[s]
        for gi, (off, sz) in enumerate(((0, 16), (16, 16), (32, 8))):
            def p1(ei, carry3):
                e = off + ei
                p0 = qbuf[e, pl.ds(0, 16)] * kbuf[e, pl.ds(0, 16)]
                p1_ = qbuf[e, pl.ds(16, 16)] * kbuf[e, pl.ds(16, 16)]
                p2 = qbuf[e, pl.ds(32, 16)] * kbuf[e, pl.ds(32, 16)]
                p3_ = qbuf[e, pl.ds(48, 16)] * kbuf[e, pl.ds(48, 16)]
                p4 = qbuf[e, pl.ds(64, 16)] * kbuf[e, pl.ds(64, 16)]
                p5 = qbuf[e, pl.ds(80, 16)] * kbuf[e, pl.ds(80, 16)]
                p6 = qbuf[e, pl.ds(96, 16)] * kbuf[e, pl.ds(96, 16)]
                p7 = qbuf[e, pl.ds(112, 16)] * kbuf[e, pl.ds(112, 16)]
                ebuf[ei, pl.ds(0, 16)] = (
                    ((p0 + p1_) + (p2 + p3_)) + ((p4 + p5) + (p6 + p7)))
                return carry3

            lax.fori_loop(0, sz, p1, 0, unroll=8 if sz == 16 else 4)

            cols = [plsc.load_gather(
                ebuf, [lanes, jnp.full((16,), l, jnp.int32)])
                for l in range(16)]
            t0 = ((cols[0] + cols[1]) + (cols[2] + cols[3]))
            t1 = ((cols[4] + cols[5]) + (cols[6] + cols[7]))
            t2 = ((cols[8] + cols[9]) + (cols[10] + cols[11]))
            t3 = ((cols[12] + cols[13]) + (cols[14] + cols[15]))
            exv = jnp.exp(((t0 + t1) + (t2 + t3)) * _INV_SQRT_D)
            evec[pl.ds(0, 16)] = exv
            dst16 = dsts[gi]
            if sz == 16:
                plsc.addupdate_scatter(dtile, [dst16], exv)
            else:
                plsc.addupdate_scatter(dtile, [dst16], exv, mask=lanes < sz)

            def p3(ei, carry3):
                e = off + ei
                exb = plsc.load_gather(
                    evec, [jnp.full((16,), ei, jnp.int32)])
                for j in range(8):
                    vbuf[e, pl.ds(16 * j, 16)] = (
                        vbuf[e, pl.ds(16 * j, 16)] * exb)
                return carry3

            lax.fori_loop(0, sz, p3, 0, unroll=8 if sz == 16 else 4)

    def half(i, b):
        if b == 0:
            with_s1 = lambda f: pl.when(i >= 1)(f)
            with_s23 = lambda f: f()
            with_s6 = lambda f: pl.when(i < 124)(f)
        else:
            with_s1 = lambda f: f()
            with_s23 = lambda f: pl.when(i < 124)(f)
            with_s6 = lambda f: pl.when(i < 124)(f)
        c = 2 * i + b
        with_s1(lambda: scatter_wait(1 - b))
        with_s23(lambda: idx_wait(1 - b))
        with_s23(lambda: gathers_issue(1 - b))
        gathers_wait(b)
        d0 = dstv[b][pl.ds(0, 16)]
        d1 = dstv[b][pl.ds(16, 16)]
        d2 = dstv[b][pl.ds(32, 16)]
        dstc[b][pl.ds(0, 16)] = d0
        dstc[b][pl.ds(16, 16)] = d1
        plsc.store_scatter(dstc[b], [lanes + 32], d2, mask=lanes < 8)
        with_s6(lambda: idx_fetch(c + 2, b, sync=False))
        compute(b, (d0, d1, d2))
        scatter_issue(b)

    idx_fetch(0, 0, sync=True)
    gathers_issue(0)
    idx_fetch(1, 1, sync=False)

    def step(i, carry):
        half(i, 0)
        half(i, 1)
        return carry

    lax.fori_loop(0, _CPT // 2, step, 0)
    scatter_wait(1)

    plsc.subcore_barrier()
    pltpu.sync_copy(acc.at[pl.ds(sid * _RPT, _RPT)],
                    part_hbm.at[cid, pl.ds(sid * _RPT, _RPT)])
    pltpu.sync_copy(dtile, dpart_hbm.at[cid, sid])


@functools.cache
def _make_edge():
    mesh = plsc.VectorSubcoreMesh(
        core_axis_name="c", subcore_axis_name="s",
        num_cores=_NC, num_subcores=_NS)
    return pl.kernel(
        _edge_body,
        out_type=(jax.ShapeDtypeStruct((_NC, _NP, _D), jnp.float32),
                  jax.ShapeDtypeStruct((_NC, _NS, _NP), jnp.float32)),
        mesh=mesh,
        compiler_params=pltpu.CompilerParams(needs_layout_passes=False),
        scratch_types=[
            pltpu.VMEM((48,), jnp.int32),
            pltpu.VMEM((48,), jnp.int32),
            pltpu.VMEM((48,), jnp.int32),
            pltpu.VMEM((48,), jnp.int32),
            pltpu.VMEM((_C,), jnp.int32),
            pltpu.VMEM((_C,), jnp.int32),
            pltpu.VMEM((_C, _D), jnp.float32),
            pltpu.VMEM((_C, _D), jnp.float32),
            pltpu.VMEM((_C, _D), jnp.float32),
            pltpu.VMEM((_C, _D), jnp.float32),
            pltpu.VMEM((_C, _D), jnp.float32),
            pltpu.VMEM((_C, _D), jnp.float32),
            pltpu.VMEM((16, 16), jnp.float32),
            pltpu.VMEM((16,), jnp.float32),
            pltpu.VMEM((_NP,), jnp.float32),
            pltpu.VMEM_SHARED((_NP, _D), jnp.float32),
            pltpu.SemaphoreType.DMA,
            pltpu.SemaphoreType.DMA,
            pltpu.SemaphoreType.DMA,
            pltpu.SemaphoreType.DMA,
        ],
    )


def _edge(q, k, v, src, dst, zn, zd):
    return _make_edge()(q, k, v, src, dst, zn, zd)


def kernel(x, edge_index, W1q, b1q, W1k, b1k, W1v, b1v, W1s, b1s,
           W2q, b2q, W2k, b2k, W2v, b2v, W2s, b2s):
    src = edge_index[0]
    dst = edge_index[1]
    w1 = jnp.concatenate([W1q.T, W1k.T, W1v.T, W1s.T], axis=1)
    b1 = jnp.concatenate([b1q, b1k, b1v, b1s])[None, :]
    w2 = jnp.concatenate([W2q.T, W2k.T, W2v.T, W2s.T], axis=1)
    b2 = jnp.concatenate([b2q, b2k, b2v, b2s])[None, :]
    xp = jnp.zeros((_NP, _D), jnp.float32).at[:_N].set(x)
    zn = jnp.zeros((_RPT, _D), jnp.float32)
    zd = jnp.zeros((_NP,), jnp.float32)
    ones = jnp.ones((_NS, 1), jnp.float32)

    q1, k1, v1, s1 = _proj(xp, w1, b1)
    p1, dp1 = _edge(q1, k1, v1, src, dst, zn, zd)
    q2, k2, v2, s2 = _comb_proj(p1, dp1, s1, ones, w2, b2)
    p2, dp2 = _edge(q2, k2, v2, src, dst, zn, zd)
    return _comb(p2, dp2, s2, ones)[:_N]


# speedup vs baseline: 12.9767x; 1.0030x over previous
import functools

import jax
import jax.numpy as jnp
from jax import lax
from jax.experimental import pallas as pl
from jax.experimental.pallas import tpu as pltpu
from jax.experimental.pallas import tpu_sc as plsc

_N = 10000
_E = 320000
_D = 128
_NC = 2
_NS = 16
_NW = _NC * _NS
_EPW = _E // _NW
_C = 40
_NP = 10240
_RPT = _NP // _NS
_INV_SQRT_D = 1.0 / (_D ** 0.5)

_BN = 640
_GRID = _NP // _BN


def _proj_body(x_ref, w_ref, b_ref, q_ref, k_ref, v_ref, s_ref):
    acc = jnp.dot(x_ref[...], w_ref[...],
                  preferred_element_type=jnp.float32) + b_ref[...]
    q_ref[...] = acc[:, :_D]
    k_ref[...] = acc[:, _D:2 * _D]
    v_ref[...] = acc[:, 2 * _D:3 * _D]
    s_ref[...] = acc[:, 3 * _D:]


def _qkvs_specs():
    return dict(
        out_specs=[
            pl.BlockSpec((_BN, _D), lambda i: (i, 0)),
            pl.BlockSpec((_BN, _D), lambda i: (i, 0)),
            pl.BlockSpec((_BN, _D), lambda i: (i, 0)),
            pl.BlockSpec((_BN, _D), lambda i: (i, 0)),
        ],
        out_shape=[
            jax.ShapeDtypeStruct((_NP, _D), jnp.float32),
            jax.ShapeDtypeStruct((_NP, _D), jnp.float32),
            jax.ShapeDtypeStruct((_NP, _D), jnp.float32),
            jax.ShapeDtypeStruct((_NP, _D), jnp.float32),
        ],
    )


def _proj(x, w, b):
    return pl.pallas_call(
        _proj_body,
        grid=(_GRID,),
        in_specs=[
            pl.BlockSpec((_BN, _D), lambda i: (i, 0)),
            pl.BlockSpec((_D, 4 * _D), lambda i: (0, 0)),
            pl.BlockSpec((1, 4 * _D), lambda i: (0, 0)),
        ],
        **_qkvs_specs(),
    )(x, w, b)


def _combine_h(p_ref, dp_ref, s_ref, ones_ref):
    num = p_ref[0] + p_ref[1]
    dsum = dp_ref[0] + dp_ref[1]
    den = lax.dot_general(dsum, ones_ref[...],
                          (((0,), (0,)), ((), ())),
                          preferred_element_type=jnp.float32)
    return num / (den + 1e-16) + s_ref[...]


def _comb_proj_body(p_ref, dp_ref, s1_ref, ones_ref, w_ref, b_ref,
                    q_ref, k_ref, v_ref, s_ref):
    h = jnp.maximum(_combine_h(p_ref, dp_ref, s1_ref, ones_ref), 0.0)
    acc = jnp.dot(h, w_ref[...],
                  preferred_element_type=jnp.float32) + b_ref[...]
    q_ref[...] = acc[:, :_D]
    k_ref[...] = acc[:, _D:2 * _D]
    v_ref[...] = acc[:, 2 * _D:3 * _D]
    s_ref[...] = acc[:, 3 * _D:]


def _comb_proj(p, dp, s1, ones, w, b):
    return pl.pallas_call(
        _comb_proj_body,
        grid=(_GRID,),
        in_specs=[
            pl.BlockSpec((2, _BN, _D), lambda i: (0, i, 0)),
            pl.BlockSpec((2, _NS, _BN), lambda i: (0, 0, i)),
            pl.BlockSpec((_BN, _D), lambda i: (i, 0)),
            pl.BlockSpec((_NS, 1), lambda i: (0, 0)),
            pl.BlockSpec((_D, 4 * _D), lambda i: (0, 0)),
            pl.BlockSpec((1, 4 * _D), lambda i: (0, 0)),
        ],
        **_qkvs_specs(),
    )(p, dp, s1, ones, w, b)


def _comb_body(p_ref, dp_ref, s2_ref, ones_ref, o_ref):
    o_ref[...] = _combine_h(p_ref, dp_ref, s2_ref, ones_ref)


def _comb(p, dp, s2, ones):
    return pl.pallas_call(
        _comb_body,
        grid=(_GRID,),
        in_specs=[
            pl.BlockSpec((2, _BN, _D), lambda i: (0, i, 0)),
            pl.BlockSpec((2, _NS, _BN), lambda i: (0, 0, i)),
            pl.BlockSpec((_BN, _D), lambda i: (i, 0)),
            pl.BlockSpec((_NS, 1), lambda i: (0, 0)),
        ],
        out_specs=pl.BlockSpec((_BN, _D), lambda i: (i, 0)),
        out_shape=jax.ShapeDtypeStruct((_NP, _D), jnp.float32),
    )(p, dp, s2, ones)


_CPT = 250


def _edge_body(q_hbm, k_hbm, v_hbm, src_hbm, dst_hbm, zn_hbm, zd_hbm,
               part_hbm, dpart_hbm,
               srcv0, srcv1, dstv0, dstv1, dstc0, dstc1,
               qb0, qb1, kb0, kb1, vb0, vb1, ebuf, evec, dtile,
               acc, semg0, semg1, semi, sems):
    srcv = (srcv0, srcv1)
    dstv = (dstv0, dstv1)
    dstc = (dstc0, dstc1)
    qb = (qb0, qb1)
    kb = (kb0, kb1)
    vb = (vb0, vb1)
    semg = (semg0, semg1)

    cid = lax.axis_index("c")
    sid = lax.axis_index("s")
    wid = sid * _NC + cid

    pltpu.sync_copy(zn_hbm, acc.at[pl.ds(sid * _RPT, _RPT)])
    pltpu.sync_copy(zd_hbm, dtile)
    plsc.subcore_barrier()

    lanes = lax.iota(jnp.int32, 16)
    base_w = wid * _EPW

    def idx_fetch(c, s, sync):
        base = base_w + c * _C
        if sync:
            pltpu.sync_copy(src_hbm.at[pl.ds(base, _C)],
                            srcv[s].at[pl.ds(0, _C)])
            pltpu.sync_copy(dst_hbm.at[pl.ds(base, _C)],
                            dstv[s].at[pl.ds(0, _C)])
        else:
            pltpu.async_copy(src_hbm.at[pl.ds(base, _C)],
                             srcv[s].at[pl.ds(0, _C)], semi)
            pltpu.async_copy(dst_hbm.at[pl.ds(base, _C)],
                             dstv[s].at[pl.ds(0, _C)], semi)

    def idx_wait(s):
        pltpu.make_async_copy(src_hbm.at[pl.ds(0, _C)],
                              srcv[s].at[pl.ds(0, _C)], semi).wait()
        pltpu.make_async_copy(dst_hbm.at[pl.ds(0, _C)],
                              dstv[s].at[pl.ds(0, _C)], semi).wait()

    def gathers_issue(s):
        pltpu.async_copy(k_hbm.at[srcv[s].at[pl.ds(0, _C)]], kb[s], semg[s])
        pltpu.async_copy(v_hbm.at[srcv[s].at[pl.ds(0, _C)]], vb[s], semg[s])
        pltpu.async_copy(q_hbm.at[dstv[s].at[pl.ds(0, _C)]], qb[s], semg[s])

    def gathers_wait(s):
        for ref in (kb[s], vb[s], qb[s]):
            pltpu.make_async_copy(
                k_hbm.at[srcv[s].at[pl.ds(0, _C)]], ref, semg[s]).wait()

    def scatter_issue(s):
        return pltpu.async_copy(vb[s], acc.at[dstc[s]], sems, add=True)

    def scatter_wait(s):
        pltpu.make_async_copy(vb[s], acc.at[dstc[s]], sems).wait()

    def compute(s, dsts):
        qbuf, kbuf, vbuf = qb[s], kb[s], vb[s]
        for gi, (off, sz) in enumerate(((0, 16), (16, 16), (32, 8))):
            def p1(ei, carry3):
                e = off + ei
                p0 = qbuf[e, pl.ds(0, 16)] * kbuf[e, pl.ds(0, 16)]
                p1_ = qbuf[e, pl.ds(16, 16)] * kbuf[e, pl.ds(16, 16)]
                p2 = qbuf[e, pl.ds(32, 16)] * kbuf[e, pl.ds(32, 16)]
                p3_ = qbuf[e, pl.ds(48, 16)] * kbuf[e, pl.ds(48, 16)]
                p4 = qbuf[e, pl.ds(64, 16)] * kbuf[e, pl.ds(64, 16)]
                p5 = qbuf[e, pl.ds(80, 16)] * kbuf[e, pl.ds(80, 16)]
                p6 = qbuf[e, pl.ds(96, 16)] * kbuf[e, pl.ds(96, 16)]
                p7 = qbuf[e, pl.ds(112, 16)] * kbuf[e, pl.ds(112, 16)]
                ebuf[ei, pl.ds(0, 16)] = (
                    ((p0 + p1_) + (p2 + p3_)) + ((p4 + p5) + (p6 + p7)))
                return carry3

            lax.fori_loop(0, sz, p1, 0, unroll=8 if sz == 16 else 4)

            cols = [plsc.load_gather(
                ebuf, [lanes, jnp.full((16,), l, jnp.int32)])
                for l in range(16)]
            t0 = ((cols[0] + cols[1]) + (cols[2] + cols[3]))
            t1 = ((cols[4] + cols[5]) + (cols[6] + cols[7]))
            t2 = ((cols[8] + cols[9]) + (cols[10] + cols[11]))
            t3 = ((cols[12] + cols[13]) + (cols[14] + cols[15]))
            exv = jnp.exp(((t0 + t1) + (t2 + t3)) * _INV_SQRT_D)
            evec[pl.ds(0, 16)] = exv
            dst16 = dsts[gi]
            if sz == 16:
                plsc.addupdate_scatter(dtile, [dst16], exv)
            else:
                plsc.addupdate_scatter(dtile, [dst16], exv, mask=lanes < sz)

            def p3(ei, carry3):
                e = off + ei
                exb = plsc.load_gather(
                    evec, [jnp.full((16,), ei, jnp.int32)])
                for j in range(8):
                    vbuf[e, pl.ds(16 * j, 16)] = (
                        vbuf[e, pl.ds(16 * j, 16)] * exb)
                return carry3

            lax.fori_loop(0, sz, p3, 0, unroll=8 if sz == 16 else 4)

    def half(i, b):
        if b == 0:
            with_s1 = lambda f: pl.when(i >= 1)(f)
            with_s23 = lambda f: f()
            with_s6 = lambda f: pl.when(i < 124)(f)
        else:
            with_s1 = lambda f: f()
            with_s23 = lambda f: pl.when(i < 124)(f)
            with_s6 = lambda f: pl.when(i < 124)(f)
        c = 2 * i + b
        with_s1(lambda: scatter_wait(1 - b))
        with_s23(lambda: idx_wait(1 - b))
        with_s23(lambda: gathers_issue(1 - b))
        gathers_wait(b)
        d0 = dstv[b][pl.ds(0, 16)]
        d1 = dstv[b][pl.ds(16, 16)]
        d2 = dstv[b][pl.ds(32, 16)]
        dstc[b][pl.ds(0, 16)] = d0
        dstc[b][pl.ds(16, 16)] = d1
        plsc.store_scatter(dstc[b], [lanes + 32], d2, mask=lanes < 8)
        with_s6(lambda: idx_fetch(c + 2, b, sync=False))
        compute(b, (d0, d1, d2))
        scatter_issue(b)

    idx_fetch(0, 0, sync=True)
    gathers_issue(0)
    idx_fetch(1, 1, sync=False)

    def step(i, carry):
        half(i, 0)
        half(i, 1)
        return carry

    lax.fori_loop(0, _CPT // 2, step, 0)
    scatter_wait(1)

    plsc.subcore_barrier()
    pltpu.sync_copy(acc.at[pl.ds(sid * _RPT, _RPT)],
                    part_hbm.at[cid, pl.ds(sid * _RPT, _RPT)])
    pltpu.sync_copy(dtile, dpart_hbm.at[cid, sid])


@functools.cache
def _make_edge():
    mesh = plsc.VectorSubcoreMesh(
        core_axis_name="c", subcore_axis_name="s",
        num_cores=_NC, num_subcores=_NS)
    return pl.kernel(
        _edge_body,
        out_type=(jax.ShapeDtypeStruct((_NC, _NP, _D), jnp.float32),
                  jax.ShapeDtypeStruct((_NC, _NS, _NP), jnp.float32)),
        mesh=mesh,
        compiler_params=pltpu.CompilerParams(needs_layout_passes=False),
        scratch_types=[
            pltpu.VMEM((48,), jnp.int32),
            pltpu.VMEM((48,), jnp.int32),
            pltpu.VMEM((48,), jnp.int32),
            pltpu.VMEM((48,), jnp.int32),
            pltpu.VMEM((_C,), jnp.int32),
            pltpu.VMEM((_C,), jnp.int32),
            pltpu.VMEM((_C, _D), jnp.float32),
            pltpu.VMEM((_C, _D), jnp.float32),
            pltpu.VMEM((_C, _D), jnp.float32),
            pltpu.VMEM((_C, _D), jnp.float32),
            pltpu.VMEM((_C, _D), jnp.float32),
            pltpu.VMEM((_C, _D), jnp.float32),
            pltpu.VMEM((16, 16), jnp.float32),
            pltpu.VMEM((16,), jnp.float32),
            pltpu.VMEM((_NP,), jnp.float32),
            pltpu.VMEM_SHARED((_NP, _D), jnp.float32),
            pltpu.SemaphoreType.DMA,
            pltpu.SemaphoreType.DMA,
            pltpu.SemaphoreType.DMA,
            pltpu.SemaphoreType.DMA,
        ],
    )


def _edge(q, k, v, src, dst, zn, zd):
    return _make_edge()(q, k, v, src, dst, zn, zd)


def kernel(x, edge_index, W1q, b1q, W1k, b1k, W1v, b1v, W1s, b1s,
           W2q, b2q, W2k, b2k, W2v, b2v, W2s, b2s):
    src = edge_index[0]
    dst = edge_index[1]
    w1 = jnp.concatenate([W1q.T, W1k.T, W1v.T, W1s.T], axis=1)
    b1 = jnp.concatenate([b1q, b1k, b1v, b1s])[None, :]
    w2 = jnp.concatenate([W2q.T, W2k.T, W2v.T, W2s.T], axis=1)
    b2 = jnp.concatenate([b2q, b2k, b2v, b2s])[None, :]
    xp = jnp.zeros((_NP, _D), jnp.float32).at[:_N].set(x)
    zn = jnp.zeros((_RPT, _D), jnp.float32)
    zd = jnp.zeros((_NP,), jnp.float32)
    ones = jnp.ones((_NS, 1), jnp.float32)

    q1, k1, v1, s1 = _proj(xp, w1, b1)
    p1, dp1 = _edge(q1, k1, v1, src, dst, zn, zd)
    q2, k2, v2, s2 = _comb_proj(p1, dp1, s1, ones, w2, b2)
    p2, dp2 = _edge(q2, k2, v2, src, dst, zn, zd)
    return _comb(p2, dp2, s2, ones)[:_N]
